# baseline (speedup 1.0000x reference)
import functools

import jax
import jax.numpy as jnp
from jax.experimental import pallas as pl
from jax.experimental.pallas import tpu as pltpu


def _nbeats_kernel(x_ref, w1_ref, b1_ref, w2_ref, b2_ref, w3t_ref, b3_ref,
                   o_ref, *, nb):
    inp = x_ref[...]
    b3s = jnp.sum(b3_ref[...], axis=0, keepdims=True)
    out_t = jnp.swapaxes(b3s, 0, 1)
    hid = w1_ref.shape[-1]
    w1cat = jnp.concatenate(
        [w1_ref[b].astype(jnp.bfloat16) for b in range(nb)], axis=1)
    b1cat = jnp.concatenate([b1_ref[b:b + 1, :] for b in range(nb)], axis=1)
    h1 = jnp.dot(inp, w1cat, preferred_element_type=jnp.float32)
    h1 = jnp.maximum(h1 + b1cat, 0.0).astype(jnp.bfloat16)
    hs = []
    for b in range(nb):
        h = jnp.dot(h1[:, b * hid:(b + 1) * hid],
                    w2_ref[b].astype(jnp.bfloat16),
                    preferred_element_type=jnp.float32)
        hs.append(jnp.maximum(h + b2_ref[b:b + 1, :], 0.0).astype(jnp.bfloat16))
    hcat = jnp.concatenate(hs, axis=1)
    w3cat = jnp.concatenate(
        [w3t_ref[b].astype(jnp.bfloat16) for b in range(nb)], axis=1)
    out_t = out_t + jnp.einsum("mk,nk->mn", w3cat, hcat,
                               preferred_element_type=jnp.float32)
    o_ref[...] = out_t


def kernel(x, w1, b1, w2, b2, w3, b3):
    B, t_in, nf = x.shape
    nb, _, hid = w1.shape
    t_out = w3.shape[-1]

    inp = x[:, :, -1].astype(jnp.bfloat16)
    w3t = jnp.swapaxes(w3, 1, 2)

    tb = 1024 if B % 1024 == 0 else B
    out_t = pl.pallas_call(
        functools.partial(_nbeats_kernel, nb=nb),
        out_shape=jax.ShapeDtypeStruct((t_out, B), jnp.float32),
        grid=(B // tb,),
        in_specs=[
            pl.BlockSpec((tb, t_in), lambda i: (i, 0)),
            pl.BlockSpec(w1.shape, lambda i: (0, 0, 0)),
            pl.BlockSpec(b1.shape, lambda i: (0, 0)),
            pl.BlockSpec(w2.shape, lambda i: (0, 0, 0)),
            pl.BlockSpec(b2.shape, lambda i: (0, 0)),
            pl.BlockSpec((nb, t_out, hid), lambda i: (0, 0, 0)),
            pl.BlockSpec(b3.shape, lambda i: (0, 0)),
        ],
        out_specs=pl.BlockSpec((t_out, tb), lambda i: (0, i)),
        compiler_params=pltpu.CompilerParams(
            dimension_semantics=("parallel",)),
    )(inp, w1, b1, w2, b2, w3t, b3)
    return out_t.T


# speedup vs baseline: 1.0666x; 1.0666x over previous
import functools

import jax
import jax.numpy as jnp
from jax.experimental import pallas as pl
from jax.experimental.pallas import tpu as pltpu


def _nbeats_kernel(x_ref, w1_ref, b1_ref, w2_ref, b2_ref, w3t_ref, b3_ref,
                   o_ref, *, nb):
    inp = x_ref[...]
    b3s = jnp.sum(b3_ref[...], axis=0, keepdims=True)
    out_t = jnp.swapaxes(b3s, 0, 1)
    hid = w1_ref.shape[-1]
    w1cat = jnp.concatenate(
        [w1_ref[b].astype(jnp.bfloat16) for b in range(nb)], axis=1)
    b1cat = jnp.concatenate([b1_ref[b:b + 1, :] for b in range(nb)], axis=1)
    h1 = jnp.dot(inp, w1cat, preferred_element_type=jnp.float32)
    h1 = jnp.maximum(h1 + b1cat, 0.0).astype(jnp.bfloat16)
    hs = []
    for b in range(nb):
        h = jnp.dot(h1[:, b * hid:(b + 1) * hid],
                    w2_ref[b].astype(jnp.bfloat16),
                    preferred_element_type=jnp.float32)
        hs.append(jnp.maximum(h + b2_ref[b:b + 1, :], 0.0).astype(jnp.bfloat16))
    hcat = jnp.concatenate(hs, axis=1)
    w3cat = jnp.concatenate(
        [w3t_ref[b].astype(jnp.bfloat16) for b in range(nb)], axis=1)
    out_t = out_t + jnp.einsum("mk,nk->mn", w3cat, hcat,
                               preferred_element_type=jnp.float32)
    o_ref[...] = out_t


def kernel(x, w1, b1, w2, b2, w3, b3):
    B, t_in, nf = x.shape
    nb, _, hid = w1.shape
    t_out = w3.shape[-1]

    inp = x[:, :, -1].astype(jnp.bfloat16)
    w3t = jnp.swapaxes(w3, 1, 2)

    tb = 2048 if B % 2048 == 0 else B
    out_t = pl.pallas_call(
        functools.partial(_nbeats_kernel, nb=nb),
        out_shape=jax.ShapeDtypeStruct((t_out, B), jnp.float32),
        grid=(B // tb,),
        in_specs=[
            pl.BlockSpec((tb, t_in), lambda i: (i, 0)),
            pl.BlockSpec(w1.shape, lambda i: (0, 0, 0)),
            pl.BlockSpec(b1.shape, lambda i: (0, 0)),
            pl.BlockSpec(w2.shape, lambda i: (0, 0, 0)),
            pl.BlockSpec(b2.shape, lambda i: (0, 0)),
            pl.BlockSpec((nb, t_out, hid), lambda i: (0, 0, 0)),
            pl.BlockSpec(b3.shape, lambda i: (0, 0)),
        ],
        out_specs=pl.BlockSpec((t_out, tb), lambda i: (0, i)),
        compiler_params=pltpu.CompilerParams(
            dimension_semantics=("parallel",)),
    )(inp, w1, b1, w2, b2, w3t, b3)
    return out_t.T
